# baseline (speedup 1.0000x reference)
import jax
import jax.numpy as jnp
from jax import lax
from jax.experimental import pallas as pl
from jax.experimental.pallas import tpu as pltpu
from jax.experimental.pallas import tpu_sc as plsc

VOCAB, DIM, B, C, NEG = 1_000_000, 64, 16384, 20, 20
L = 16
NW = 2 * 16
BPW = B // NW
NB = 32
NCHUNK = BPW // NB
GP = NB * C // 128
DK = DIM // L
TW = 16384


def _sc_body(u_tab, v_tab, u_pos, v_posf, v_negf,
             out_pos, out_neg,
             idx_u, idx_p, idx_n, rows_u, rows_p, rows_n,
             part_p, part_n, sem):
    cid = lax.axis_index("c")
    sid = lax.axis_index("s")
    wid = cid * 16 + sid

    def chunk_body(ch, carry):
        base = wid * BPW + ch * NB
        pltpu.sync_copy(u_pos.at[pl.ds(base, NB)], idx_u)
        pltpu.sync_copy(v_posf.at[pl.ds(base * C, NB * C)], idx_p)
        pltpu.sync_copy(v_negf.at[pl.ds(base * NEG, NB * NEG)], idx_n)
        cps = [pltpu.async_copy(u_tab.at[idx_u], rows_u, sem)]
        for i in range(GP):
            cps.append(pltpu.async_copy(
                v_tab.at[idx_p.at[pl.ds(i * 128, 128)]],
                rows_p.at[pl.ds(i * 128, 128)], sem))
            cps.append(pltpu.async_copy(
                v_tab.at[idx_n.at[pl.ds(i * 128, 128)]],
                rows_n.at[pl.ds(i * 128, 128)], sem))
        for cp in cps:
            cp.wait()

        def b_body(b, carry2):
            u = [rows_u[b, pl.ds(k * L, L)] for k in range(DK)]
            r0 = b * C
            accp = [rows_p[r0, pl.ds(k * L, L)] for k in range(DK)]
            for c in range(1, C):
                for k in range(DK):
                    accp[k] = accp[k] + rows_p[r0 + c, pl.ds(k * L, L)]
            tp = accp[0] * u[0]
            for k in range(1, DK):
                tp = tp + accp[k] * u[k]
            part_p[b, :] = tp
            r1 = b * NEG
            accn = [rows_n[r1, pl.ds(k * L, L)] for k in range(DK)]
            for c in range(1, NEG):
                for k in range(DK):
                    accn[k] = accn[k] + rows_n[r1 + c, pl.ds(k * L, L)]
            tn = accn[0] * u[0]
            for k in range(1, DK):
                tn = tn + accn[k] * u[k]
            part_n[b, :] = tn
            return carry2

        lax.fori_loop(0, NB, b_body, 0)
        pltpu.sync_copy(part_p, out_pos.at[pl.ds(base, NB)])
        pltpu.sync_copy(part_n, out_neg.at[pl.ds(base, NB)])
        return carry

    lax.fori_loop(0, NCHUNK, chunk_body, 0)


def _transpose_body(tin_ref, tout_ref):
    tout_ref[...] = tin_ref[...].T


def _to_row_major(table_t):
    return pl.pallas_call(
        _transpose_body,
        grid=(pl.cdiv(VOCAB, TW),),
        in_specs=[pl.BlockSpec((DIM, TW), lambda j: (0, j))],
        out_specs=pl.BlockSpec((TW, DIM), lambda j: (j, 0)),
        out_shape=jax.ShapeDtypeStruct((VOCAB, DIM), jnp.float32),
        compiler_params=pltpu.CompilerParams(
            vmem_limit_bytes=56 * 1024 * 1024),
    )(table_t)


def _finish_body(pos_ref, neg_ref, out_ref):
    sp = jnp.sum(pos_ref[...], axis=1, keepdims=True)
    sn = jnp.sum(neg_ref[...], axis=1, keepdims=True)

    def log_sigmoid(x):
        return jnp.minimum(x, 0.0) - jnp.log1p(jnp.exp(-jnp.abs(x)))

    out_ref[0, 0] = -jnp.sum(log_sigmoid(sp) + log_sigmoid(-sn)) / B


def kernel(u_table, v_table, u_pos, v_pos, v_neg):
    u_pos = u_pos.astype(jnp.int32)
    v_posf = v_pos.astype(jnp.int32).reshape(B * C)
    v_negf = v_neg.astype(jnp.int32).reshape(B * NEG)
    u_rm = _to_row_major(u_table.T)
    v_rm = _to_row_major(v_table.T)

    sc = pl.kernel(
        _sc_body,
        out_type=(jax.ShapeDtypeStruct((B, L), jnp.float32),
                  jax.ShapeDtypeStruct((B, L), jnp.float32)),
        mesh=plsc.VectorSubcoreMesh(core_axis_name="c", subcore_axis_name="s"),
        scratch_types=[
            pltpu.VMEM((NB,), jnp.int32),
            pltpu.VMEM((NB * C,), jnp.int32),
            pltpu.VMEM((NB * NEG,), jnp.int32),
            pltpu.VMEM((NB, DIM), jnp.float32),
            pltpu.VMEM((NB * C, DIM), jnp.float32),
            pltpu.VMEM((NB * NEG, DIM), jnp.float32),
            pltpu.VMEM((NB, L), jnp.float32),
            pltpu.VMEM((NB, L), jnp.float32),
            pltpu.SemaphoreType.DMA,
        ],
        compiler_params=pltpu.CompilerParams(use_tc_tiling_on_sc=False),
    )
    part_pos, part_neg = sc(u_rm, v_rm, u_pos, v_posf, v_negf)

    loss = pl.pallas_call(
        _finish_body,
        out_shape=jax.ShapeDtypeStruct((1, 1), jnp.float32),
        out_specs=pl.BlockSpec(memory_space=pltpu.SMEM),
    )(part_pos, part_neg)
    return loss[0, 0]


# speedup vs baseline: 1.6739x; 1.6739x over previous
import jax
import jax.numpy as jnp
from jax import lax
from jax.experimental import pallas as pl
from jax.experimental.pallas import tpu as pltpu
from jax.experimental.pallas import tpu_sc as plsc

VOCAB, DIM, B, C, NEG = 1_000_000, 64, 16384, 20, 20
L = 16
NW = 2 * 16
BPW = B // NW
NB = 16
NCHUNK = BPW // NB
GROUPS = ((0, 128), (128, 128), (256, 64))
DK = DIM // L
TW = 16384


def _sc_body(u_tab, v_tab, u_pos, v_posf, v_negf,
             out_pos, out_neg,
             idx_u, idx_p, idx_n, rows_u, rows_p, rows_n,
             part_p, part_n, sem):
    cid = lax.axis_index("c")
    sid = lax.axis_index("s")
    wid = cid * 16 + sid

    def chunk_body(ch, carry):
        base = wid * BPW + ch * NB
        pltpu.sync_copy(u_pos.at[pl.ds(base, NB)], idx_u)
        pltpu.sync_copy(v_posf.at[pl.ds(base * C, NB * C)], idx_p)
        pltpu.sync_copy(v_negf.at[pl.ds(base * NEG, NB * NEG)], idx_n)
        cps = [pltpu.async_copy(u_tab.at[idx_u], rows_u, sem)]
        for so, ln in GROUPS:
            cps.append(pltpu.async_copy(
                v_tab.at[idx_p.at[pl.ds(so, ln)]],
                rows_p.at[pl.ds(so, ln)], sem))
            cps.append(pltpu.async_copy(
                v_tab.at[idx_n.at[pl.ds(so, ln)]],
                rows_n.at[pl.ds(so, ln)], sem))
        for cp in cps:
            cp.wait()

        def b_body(b, carry2):
            u = [rows_u[b, pl.ds(k * L, L)] for k in range(DK)]
            r0 = b * C
            accp = [rows_p[r0, pl.ds(k * L, L)] for k in range(DK)]
            for c in range(1, C):
                for k in range(DK):
                    accp[k] = accp[k] + rows_p[r0 + c, pl.ds(k * L, L)]
            tp = accp[0] * u[0]
            for k in range(1, DK):
                tp = tp + accp[k] * u[k]
            part_p[b, :] = tp
            r1 = b * NEG
            accn = [rows_n[r1, pl.ds(k * L, L)] for k in range(DK)]
            for c in range(1, NEG):
                for k in range(DK):
                    accn[k] = accn[k] + rows_n[r1 + c, pl.ds(k * L, L)]
            tn = accn[0] * u[0]
            for k in range(1, DK):
                tn = tn + accn[k] * u[k]
            part_n[b, :] = tn
            return carry2

        lax.fori_loop(0, NB, b_body, 0)
        pltpu.sync_copy(part_p, out_pos.at[pl.ds(base, NB)])
        pltpu.sync_copy(part_n, out_neg.at[pl.ds(base, NB)])
        return carry

    lax.fori_loop(0, NCHUNK, chunk_body, 0)


def _transpose_body(tin_ref, tout_ref):
    t = tin_ref[...].T
    tout_ref[...] = jnp.concatenate([t, t], axis=1)


def _to_row_major(table_t):
    return pl.pallas_call(
        _transpose_body,
        grid=(pl.cdiv(VOCAB, TW),),
        in_specs=[pl.BlockSpec((DIM, TW), lambda j: (0, j))],
        out_specs=pl.BlockSpec((TW, 2 * DIM), lambda j: (j, 0)),
        out_shape=jax.ShapeDtypeStruct((VOCAB, 2 * DIM), jnp.float32),
        compiler_params=pltpu.CompilerParams(
            vmem_limit_bytes=56 * 1024 * 1024),
    )(table_t)


def _finish_body(pos_ref, neg_ref, out_ref):
    sp = jnp.sum(pos_ref[...], axis=1, keepdims=True)
    sn = jnp.sum(neg_ref[...], axis=1, keepdims=True)

    def log_sigmoid(x):
        return jnp.minimum(x, 0.0) - jnp.log1p(jnp.exp(-jnp.abs(x)))

    out_ref[0, 0] = -jnp.sum(log_sigmoid(sp) + log_sigmoid(-sn)) / B


def kernel(u_table, v_table, u_pos, v_pos, v_neg):
    u_pos = u_pos.astype(jnp.int32)
    v_posf = v_pos.astype(jnp.int32).reshape(B * C)
    v_negf = v_neg.astype(jnp.int32).reshape(B * NEG)
    u_rm = _to_row_major(u_table.T)
    v_rm = _to_row_major(v_table.T)

    sc = pl.kernel(
        _sc_body,
        out_type=(jax.ShapeDtypeStruct((B, L), jnp.float32),
                  jax.ShapeDtypeStruct((B, L), jnp.float32)),
        mesh=plsc.VectorSubcoreMesh(core_axis_name="c", subcore_axis_name="s"),
        scratch_types=[
            pltpu.VMEM((NB,), jnp.int32),
            pltpu.VMEM((NB * C,), jnp.int32),
            pltpu.VMEM((NB * NEG,), jnp.int32),
            pltpu.VMEM((NB, 2 * DIM), jnp.float32),
            pltpu.VMEM((NB * C, 2 * DIM), jnp.float32),
            pltpu.VMEM((NB * NEG, 2 * DIM), jnp.float32),
            pltpu.VMEM((NB, L), jnp.float32),
            pltpu.VMEM((NB, L), jnp.float32),
            pltpu.SemaphoreType.DMA,
        ],
        compiler_params=pltpu.CompilerParams(use_tc_tiling_on_sc=False),
    )
    part_pos, part_neg = sc(u_rm, v_rm, u_pos, v_posf, v_negf)

    loss = pl.pallas_call(
        _finish_body,
        out_shape=jax.ShapeDtypeStruct((1, 1), jnp.float32),
        out_specs=pl.BlockSpec(memory_space=pltpu.SMEM),
    )(part_pos, part_neg)
    return loss[0, 0]


# speedup vs baseline: 2.1943x; 1.3109x over previous
import jax
import jax.numpy as jnp
from jax import lax
from jax.experimental import pallas as pl
from jax.experimental.pallas import tpu as pltpu
from jax.experimental.pallas import tpu_sc as plsc

VOCAB, DIM, B, C, NEG = 1_000_000, 64, 16384, 20, 20
L = 16
NW = 2 * 16
BPW = B // NW
NB = 16
NCHUNK = BPW // NB
GROUPS = ((0, 128), (128, 128), (256, 64))
DK = DIM // L
TW = 16384
NTB = pl.cdiv(VOCAB, 2 * TW)
NOUT = NTB * TW


def _sc_body(u_tab, v_tab, u_pos, v_posf, v_negf,
             out_pos, out_neg,
             idx_u, idx_p, idx_n, rows_u, rows_p, rows_n,
             part_p, part_n, sem):
    cid = lax.axis_index("c")
    sid = lax.axis_index("s")
    wid = cid * 16 + sid

    def chunk_body(ch, carry):
        base = wid * BPW + ch * NB
        pltpu.sync_copy(u_pos.at[pl.ds(base, NB)], idx_u)
        pltpu.sync_copy(v_posf.at[pl.ds(base * C, NB * C)], idx_p)
        pltpu.sync_copy(v_negf.at[pl.ds(base * NEG, NB * NEG)], idx_n)
        cps = [pltpu.async_copy(u_tab.at[idx_u], rows_u, sem)]
        for so, ln in GROUPS:
            cps.append(pltpu.async_copy(
                v_tab.at[idx_p.at[pl.ds(so, ln)]],
                rows_p.at[pl.ds(so, ln)], sem))
            cps.append(pltpu.async_copy(
                v_tab.at[idx_n.at[pl.ds(so, ln)]],
                rows_n.at[pl.ds(so, ln)], sem))
        for cp in cps:
            cp.wait()

        def b_body(b, carry2):
            u = [rows_u[b, pl.ds(k * L, L)] for k in range(DK)]
            r0 = b * C
            accp = [rows_p[r0, pl.ds(k * L, L)] for k in range(DK)]
            for c in range(1, C):
                for k in range(DK):
                    accp[k] = accp[k] + rows_p[r0 + c, pl.ds(k * L, L)]
            tp = accp[0] * u[0]
            for k in range(1, DK):
                tp = tp + accp[k] * u[k]
            part_p[b, :] = tp
            r1 = b * NEG
            accn = [rows_n[r1, pl.ds(k * L, L)] for k in range(DK)]
            for c in range(1, NEG):
                for k in range(DK):
                    accn[k] = accn[k] + rows_n[r1 + c, pl.ds(k * L, L)]
            tn = accn[0] * u[0]
            for k in range(1, DK):
                tn = tn + accn[k] * u[k]
            part_n[b, :] = tn
            return carry2

        lax.fori_loop(0, NB, b_body, 0)
        pltpu.sync_copy(part_p, out_pos.at[pl.ds(base, NB)])
        pltpu.sync_copy(part_n, out_neg.at[pl.ds(base, NB)])
        return carry

    lax.fori_loop(0, NCHUNK, chunk_body, 0)


def _transpose_body(tlo_ref, thi_ref, tout_ref):
    tout_ref[...] = jnp.concatenate([tlo_ref[...].T, thi_ref[...].T],
                                    axis=1)


def _to_row_major(table_t):
    out = pl.pallas_call(
        _transpose_body,
        grid=(NTB,),
        in_specs=[pl.BlockSpec((DIM, TW), lambda j: (0, 2 * j)),
                  pl.BlockSpec((DIM, TW), lambda j: (0, 2 * j + 1))],
        out_specs=pl.BlockSpec((TW, 2 * DIM), lambda j: (j, 0)),
        out_shape=jax.ShapeDtypeStruct((NOUT, 2 * DIM), jnp.float32),
        compiler_params=pltpu.CompilerParams(
            vmem_limit_bytes=56 * 1024 * 1024),
    )(table_t, table_t)
    return out.reshape(2 * NOUT, DIM)


def _remap(idx):
    ib = idx >> 14
    io = idx & (TW - 1)
    return ((ib >> 1) << 15) + 2 * io + (ib & 1)


def _finish_body(pos_ref, neg_ref, out_ref):
    sp = jnp.sum(pos_ref[...], axis=1, keepdims=True)
    sn = jnp.sum(neg_ref[...], axis=1, keepdims=True)

    def log_sigmoid(x):
        return jnp.minimum(x, 0.0) - jnp.log1p(jnp.exp(-jnp.abs(x)))

    out_ref[0, 0] = -jnp.sum(log_sigmoid(sp) + log_sigmoid(-sn)) / B


def kernel(u_table, v_table, u_pos, v_pos, v_neg):
    u_pos = _remap(u_pos.astype(jnp.int32))
    v_posf = _remap(v_pos.astype(jnp.int32).reshape(B * C))
    v_negf = _remap(v_neg.astype(jnp.int32).reshape(B * NEG))
    u_rm = _to_row_major(u_table.T)
    v_rm = _to_row_major(v_table.T)

    sc = pl.kernel(
        _sc_body,
        out_type=(jax.ShapeDtypeStruct((B, L), jnp.float32),
                  jax.ShapeDtypeStruct((B, L), jnp.float32)),
        mesh=plsc.VectorSubcoreMesh(core_axis_name="c", subcore_axis_name="s"),
        scratch_types=[
            pltpu.VMEM((NB,), jnp.int32),
            pltpu.VMEM((NB * C,), jnp.int32),
            pltpu.VMEM((NB * NEG,), jnp.int32),
            pltpu.VMEM((NB, DIM), jnp.float32),
            pltpu.VMEM((NB * C, DIM), jnp.float32),
            pltpu.VMEM((NB * NEG, DIM), jnp.float32),
            pltpu.VMEM((NB, L), jnp.float32),
            pltpu.VMEM((NB, L), jnp.float32),
            pltpu.SemaphoreType.DMA,
        ],
        compiler_params=pltpu.CompilerParams(use_tc_tiling_on_sc=False),
    )
    part_pos, part_neg = sc(u_rm, v_rm, u_pos, v_posf, v_negf)

    loss = pl.pallas_call(
        _finish_body,
        out_shape=jax.ShapeDtypeStruct((1, 1), jnp.float32),
        out_specs=pl.BlockSpec(memory_space=pltpu.SMEM),
    )(part_pos, part_neg)
    return loss[0, 0]
